# baseline (speedup 1.0000x reference)
import math

import jax
import jax.numpy as jnp
from jax import lax
from jax.experimental import pallas as pl
from jax.experimental.pallas import tpu as pltpu
from jax.experimental.pallas import tpu_sc as plsc

NUM_CLASSES = 128
LOG_2PI = math.log(2.0 * math.pi)

NC = 2
NS = 16
CH = 64


def _scp_body(p_hbm, idx_hbm, z_hbm, segp_hbm,
              pbufa0, pbufa1, pbufb, tbuf0, tbuf1, acca, accb,
              sga0, sga1, sgi0, sgi1, sgb, ssa, ssb):
    cid = lax.axis_index("c")
    sid = lax.axis_index("s")
    d = p_hbm.shape[1]
    dq = d // (2 * NC)
    ep_t = p_hbm.shape[0] // NS
    nchunk = ep_t // CH
    c0a = cid * dq
    c0b = (NC + cid) * dq
    r0 = sid * NUM_CLASSES

    pltpu.sync_copy(z_hbm, acca.at[pl.ds(r0, NUM_CLASSES)])
    pltpu.sync_copy(z_hbm, accb.at[pl.ds(r0, NUM_CLASSES)])

    pba = (pbufa0, pbufa1)
    tb = (tbuf0, tbuf1)
    sga = (sga0, sga1)
    sgi = (sgi0, sgi1)


    def gather(buf, g, c0, s):
        e0 = sid * ep_t + g * CH
        return pltpu.async_copy(
            p_hbm.at[pl.ds(e0, CH), pl.ds(c0, dq)], buf, s)

    def gather_idx(g):
        e0 = sid * ep_t + g * CH
        return pltpu.async_copy(idx_hbm.at[pl.ds(e0, CH)], tb[g % 2],
                                sgi[g % 2])

    def scatter(buf, g, acc, s):
        return pltpu.async_copy(buf, acc.at[tb[g % 2]], s, add=True)

    ga = {0: gather(pba[0], 0, c0a, sga[0]), 1: gather(pba[1], 1, c0a, sga[1])}
    gi = {0: gather_idx(0), 1: gather_idx(1)}
    gb = {0: gather(pbufb, 0, c0b, sgb)}
    tail = []
    for g in range(nchunk):
        ga.pop(g).wait()
        gi.pop(g).wait()
        gb.pop(g).wait()
        sca = scatter(pba[g % 2], g, acca, ssa)
        scb = scatter(pbufb, g, accb, ssb)
        scb.wait()
        if g + 1 < nchunk:
            gb[g + 1] = gather(pbufb, g + 1, c0b, sgb)
        if g + 2 < nchunk:
            sca.wait()
            ga[g + 2] = gather(pba[g % 2], g + 2, c0a, sga[g % 2])
            gi[g + 2] = gather_idx(g + 2)
        else:
            tail.append(sca)
    for sca in tail:
        sca.wait()

    pltpu.sync_copy(acca.at[pl.ds(r0, NUM_CLASSES)],
                    segp_hbm.at[pl.ds(r0, NUM_CLASSES), pl.ds(c0a, dq)])
    pltpu.sync_copy(accb.at[pl.ds(r0, NUM_CLASSES)],
                    segp_hbm.at[pl.ds(r0, NUM_CLASSES), pl.ds(c0b, dq)])


def _split_dot(oh, x):
    x_hi = x.astype(jnp.bfloat16)
    x_lo = (x - x_hi.astype(jnp.float32)).astype(jnp.bfloat16)
    dn = (((0,), (0,)), ((), ()))
    hi = lax.dot_general(oh, x_hi, dn, preferred_element_type=jnp.float32)
    lo = lax.dot_general(oh, x_lo, dn, preferred_element_type=jnp.float32)
    return hi + lo


def _tc_body(t_ref, p_ref, m_ref, segpm_out, scal_out):
    p = p_ref[0]
    m = m_ref[0]
    t = t_ref[0]
    n_ex, _ = p.shape
    cls = lax.broadcasted_iota(jnp.int32, (n_ex, NUM_CLASSES), 1)
    mask = t.reshape(n_ex, 1) == cls
    oh = mask.astype(jnp.bfloat16)

    pm = p * m
    segpm_out[0] = _split_dot(oh, pm)

    r_pmm = jnp.sum(pm * m, axis=1, keepdims=True)
    r_lp = jnp.sum(jnp.log(p), axis=1, keepdims=True)
    seg_pmm = jnp.sum(jnp.where(mask, r_pmm, 0.0), axis=0, keepdims=True)
    seg_rlp = jnp.sum(jnp.where(mask, r_lp, 0.0), axis=0, keepdims=True)
    counts = jnp.sum(mask.astype(jnp.float32), axis=0, keepdims=True)
    zero5 = jnp.zeros((5, NUM_CLASSES), jnp.float32)
    scal_out[0] = jnp.concatenate([seg_pmm, seg_rlp, counts, zero5], axis=0)


def _fin_body(segp_ref, segpm_ref, scal_ref, pm_out, ln_out):
    sp = segp_ref[0]
    spm = segpm_ref[0]
    sc = scal_ref[0]
    d = sp.shape[1]
    mean = spm * jnp.reciprocal(sp)
    pm_out[0] = mean
    seg_pmm = sc[0:1, :]
    seg_rlp = sc[1:2, :]
    ns = jnp.maximum(sc[2:3, :], 1.0)
    expo = 0.5 * (jnp.sum(spm * mean, axis=1).reshape(1, NUM_CLASSES)
                  - seg_pmm)
    log_det = 0.5 * (seg_rlp
                     - jnp.sum(jnp.log(sp), axis=1).reshape(1, NUM_CLASSES))
    ln_out[0] = 0.5 * (1.0 - ns) * (d * LOG_2PI) + log_det + expo


def kernel(means, precisions, targets):
    b, n, d = means.shape
    e = b * n
    rows = b * NUM_CLASSES
    dh = d // NC
    t3 = targets.reshape(b, 1, n)
    rowid = (targets + NUM_CLASSES * jnp.arange(b, dtype=jnp.int32)[:, None]
             ).reshape(e)
    p_flat = precisions.reshape(e, d)
    dq = d // (2 * NC)
    z = jnp.zeros((NUM_CLASSES, dq), jnp.float32)

    sc_scatter = pl.kernel(
        _scp_body,
        out_type=[jax.ShapeDtypeStruct((rows, d), jnp.float32)],
        mesh=plsc.VectorSubcoreMesh(core_axis_name="c", subcore_axis_name="s"),
        scratch_types=[
            pltpu.VMEM((CH, dq), jnp.float32),
            pltpu.VMEM((CH, dq), jnp.float32),
            pltpu.VMEM((CH, dq), jnp.float32),
            pltpu.VMEM((CH,), jnp.int32),
            pltpu.VMEM((CH,), jnp.int32),
            pltpu.VMEM_SHARED((rows, dq), jnp.float32),
            pltpu.VMEM_SHARED((rows, dq), jnp.float32),
            pltpu.SemaphoreType.DMA,
            pltpu.SemaphoreType.DMA,
            pltpu.SemaphoreType.DMA,
            pltpu.SemaphoreType.DMA,
            pltpu.SemaphoreType.DMA,
            pltpu.SemaphoreType.DMA,
            pltpu.SemaphoreType.DMA,
        ],
    )
    (segp,) = sc_scatter(p_flat, rowid, z)

    segpm, scal = pl.pallas_call(
        _tc_body,
        grid=(b,),
        in_specs=[
            pl.BlockSpec((1, 1, n), lambda i: (i, 0, 0)),
            pl.BlockSpec((1, n, d), lambda i: (i, 0, 0)),
            pl.BlockSpec((1, n, d), lambda i: (i, 0, 0)),
        ],
        out_specs=[
            pl.BlockSpec((1, NUM_CLASSES, d), lambda i: (i, 0, 0)),
            pl.BlockSpec((1, 8, NUM_CLASSES), lambda i: (i, 0, 0)),
        ],
        out_shape=[
            jax.ShapeDtypeStruct((b, NUM_CLASSES, d), jnp.float32),
            jax.ShapeDtypeStruct((b, 8, NUM_CLASSES), jnp.float32),
        ],
    )(t3, precisions, means)

    segp3 = segp.reshape(b, NUM_CLASSES, d)
    pm_o, ln_o = pl.pallas_call(
        _fin_body,
        grid=(b,),
        in_specs=[
            pl.BlockSpec((1, NUM_CLASSES, d), lambda i: (i, 0, 0)),
            pl.BlockSpec((1, NUM_CLASSES, d), lambda i: (i, 0, 0)),
            pl.BlockSpec((1, 8, NUM_CLASSES), lambda i: (i, 0, 0)),
        ],
        out_specs=[
            pl.BlockSpec((1, NUM_CLASSES, d), lambda i: (i, 0, 0)),
            pl.BlockSpec((1, 1, NUM_CLASSES), lambda i: (i, 0, 0)),
        ],
        out_shape=[
            jax.ShapeDtypeStruct((b, NUM_CLASSES, d), jnp.float32),
            jax.ShapeDtypeStruct((b, 1, NUM_CLASSES), jnp.float32),
        ],
    )(segp3, segpm, scal)

    return (pm_o, segp3, ln_o.reshape(b, NUM_CLASSES))
